# baseline (speedup 1.0000x reference)
import jax
import jax.numpy as jnp
from jax.experimental import pallas as pl

D = 64


def _tc_body(h_ref, t_ref, mt_ref, o_ref):
    h = h_ref[...]
    t = t_ref[...]
    pieces = [h[:, a:a + 1] * t for a in range(D)]
    o_mat = jnp.concatenate(pieces, axis=1)
    o_ref[...] = jnp.dot(o_mat, mt_ref[...], preferred_element_type=jnp.float32)


def kernel(data, E_record, M):
    idx = data.astype(jnp.int32)
    h = jnp.take(E_record, idx[0], axis=0)
    t = jnp.take(E_record, idx[1], axis=0)
    K = M.shape[0]
    B = h.shape[0]
    mt = M.reshape(K, D * D).T

    BT = 256
    out = pl.pallas_call(
        _tc_body,
        grid=(B // BT,),
        in_specs=[
            pl.BlockSpec((BT, D), lambda i: (i, 0)),
            pl.BlockSpec((BT, D), lambda i: (i, 0)),
            pl.BlockSpec((D * D, K), lambda i: (0, 0)),
        ],
        out_specs=pl.BlockSpec((BT, K), lambda i: (i, 0)),
        out_shape=jax.ShapeDtypeStruct((B, K), jnp.float32),
    )(h, t, mt)
    return out
